# baseline (speedup 1.0000x reference)
import jax
import jax.numpy as jnp
from jax.experimental import pallas as pl
from jax.experimental.pallas import tpu as pltpu

B = 8192
D_IN = 1024
D_OUT = 1024
E = 8
D_HID = 512
BM = 512


def _moe_kernel(x_ref, w0_ref, w1_ref, w2_ref, gw_ref, gb_ref, out_ref):
    xb = x_ref[...].astype(jnp.bfloat16)
    logits = (
        jnp.dot(xb, gw_ref[...], preferred_element_type=jnp.float32) + gb_ref[...]
    )
    logits = logits - jnp.max(logits, axis=-1, keepdims=True)
    p = jnp.exp(logits)
    alpha = p / jnp.sum(p, axis=-1, keepdims=True)

    acc = jnp.zeros((BM, D_OUT), jnp.float32)
    for e in range(E):
        h0 = jnp.dot(xb, w0_ref[e], preferred_element_type=jnp.float32)
        h0 = jnp.maximum(h0, 0.0).astype(jnp.bfloat16)
        h1 = jnp.dot(h0, w1_ref[e], preferred_element_type=jnp.float32)
        h1 = (jnp.maximum(h1, 0.0) * alpha[:, e : e + 1]).astype(jnp.bfloat16)
        acc = acc + jnp.dot(h1, w2_ref[e], preferred_element_type=jnp.float32)
    out_ref[...] = acc


def kernel(x, W0, W1, W2, gate_w, gate_b):
    w0b = W0.astype(jnp.bfloat16)
    w1b = W1.astype(jnp.bfloat16)
    w2b = W2.astype(jnp.bfloat16)
    gwb = gate_w.astype(jnp.bfloat16)
    gb = gate_b.reshape(1, E)
    grid = (B // BM,)
    return pl.pallas_call(
        _moe_kernel,
        grid=grid,
        in_specs=[
            pl.BlockSpec((BM, D_IN), lambda i: (i, 0)),
            pl.BlockSpec((E, D_IN, D_HID), lambda i: (0, 0, 0)),
            pl.BlockSpec((E, D_HID, D_HID), lambda i: (0, 0, 0)),
            pl.BlockSpec((E, D_HID, D_OUT), lambda i: (0, 0, 0)),
            pl.BlockSpec((D_IN, E), lambda i: (0, 0)),
            pl.BlockSpec((1, E), lambda i: (0, 0)),
        ],
        out_specs=pl.BlockSpec((BM, D_OUT), lambda i: (i, 0)),
        out_shape=jax.ShapeDtypeStruct((B, D_OUT), jnp.float32),
    )(x, w0b, w1b, w2b, gwb, gb)


# speedup vs baseline: 1.0474x; 1.0474x over previous
import jax
import jax.numpy as jnp
from jax.experimental import pallas as pl
from jax.experimental.pallas import tpu as pltpu

B = 8192
D_IN = 1024
D_OUT = 1024
E = 8
D_HID = 512
BM = 1024


def _moe_kernel(x_ref, w0_ref, w1_ref, w2_ref, gw_ref, gb_ref, out_ref, h0s, h1s):
    xb = x_ref[...].astype(jnp.bfloat16)
    logits = (
        jnp.dot(xb, gw_ref[...], preferred_element_type=jnp.float32) + gb_ref[...]
    )
    logits = logits - jnp.max(logits, axis=-1, keepdims=True)
    p = jnp.exp(logits)
    alpha = p / jnp.sum(p, axis=-1, keepdims=True)

    for e in range(E):
        h0 = jnp.dot(xb, w0_ref[e], preferred_element_type=jnp.float32)
        h0s[:, e * D_HID : (e + 1) * D_HID] = jnp.maximum(h0, 0.0).astype(
            jnp.bfloat16
        )

    for e in range(E):
        h1 = jnp.dot(
            h0s[:, e * D_HID : (e + 1) * D_HID],
            w1_ref[e],
            preferred_element_type=jnp.float32,
        )
        h1 = jnp.maximum(h1, 0.0) * alpha[:, e : e + 1]
        h1s[:, e * D_HID : (e + 1) * D_HID] = h1.astype(jnp.bfloat16)

    out_ref[...] = jnp.dot(h1s[...], w2_ref[...], preferred_element_type=jnp.float32)


def kernel(x, W0, W1, W2, gate_w, gate_b):
    w0b = W0.astype(jnp.bfloat16)
    w1b = W1.astype(jnp.bfloat16)
    w2b = W2.reshape(E * D_HID, D_OUT).astype(jnp.bfloat16)
    gwb = gate_w.astype(jnp.bfloat16)
    gb = gate_b.reshape(1, E)
    grid = (B // BM,)
    return pl.pallas_call(
        _moe_kernel,
        grid=grid,
        in_specs=[
            pl.BlockSpec((BM, D_IN), lambda i: (i, 0)),
            pl.BlockSpec((E, D_IN, D_HID), lambda i: (0, 0, 0)),
            pl.BlockSpec((E, D_HID, D_HID), lambda i: (0, 0, 0)),
            pl.BlockSpec((E * D_HID, D_OUT), lambda i: (0, 0)),
            pl.BlockSpec((D_IN, E), lambda i: (0, 0)),
            pl.BlockSpec((1, E), lambda i: (0, 0)),
        ],
        out_specs=pl.BlockSpec((BM, D_OUT), lambda i: (i, 0)),
        out_shape=jax.ShapeDtypeStruct((B, D_OUT), jnp.float32),
        scratch_shapes=[
            pltpu.VMEM((BM, E * D_HID), jnp.bfloat16),
            pltpu.VMEM((BM, E * D_HID), jnp.bfloat16),
        ],
    )(x, w0b, w1b, w2b, gwb, gb)


# speedup vs baseline: 1.0865x; 1.0373x over previous
import jax
import jax.numpy as jnp
from jax.experimental import pallas as pl
from jax.experimental.pallas import tpu as pltpu

B = 8192
D_IN = 1024
D_OUT = 1024
E = 8
D_HID = 512
BM = 512


def _moe_kernel(x_ref, w0_ref, w1_ref, w2_ref, gw_ref, gb_ref, out_ref):
    x = x_ref[...]
    logits = (
        jnp.dot(x, gw_ref[...], preferred_element_type=jnp.float32) + gb_ref[...]
    )
    logits = logits - jnp.max(logits, axis=-1, keepdims=True)
    p = jnp.exp(logits)
    alpha = p / jnp.sum(p, axis=-1, keepdims=True)

    h1s = []
    for e in range(E):
        h0 = jnp.dot(x, w0_ref[e], preferred_element_type=jnp.float32)
        h0 = jnp.maximum(h0, 0.0)
        h1 = jnp.dot(h0, w1_ref[e], preferred_element_type=jnp.float32)
        h1s.append(jnp.maximum(h1, 0.0) * alpha[:, e : e + 1])
    h1cat = jnp.concatenate(h1s, axis=1)
    out_ref[...] = jnp.dot(h1cat, w2_ref[...], preferred_element_type=jnp.float32)


def kernel(x, W0, W1, W2, gate_w, gate_b):
    w2cat = W2.reshape(E * D_HID, D_OUT)
    gb = gate_b.reshape(1, E)
    grid = (B // BM,)
    return pl.pallas_call(
        _moe_kernel,
        grid=grid,
        in_specs=[
            pl.BlockSpec((BM, D_IN), lambda i: (i, 0)),
            pl.BlockSpec((E, D_IN, D_HID), lambda i: (0, 0, 0)),
            pl.BlockSpec((E, D_HID, D_HID), lambda i: (0, 0, 0)),
            pl.BlockSpec((E * D_HID, D_OUT), lambda i: (0, 0)),
            pl.BlockSpec((D_IN, E), lambda i: (0, 0)),
            pl.BlockSpec((1, E), lambda i: (0, 0)),
        ],
        out_specs=pl.BlockSpec((BM, D_OUT), lambda i: (i, 0)),
        out_shape=jax.ShapeDtypeStruct((B, D_OUT), jnp.float32),
    )(x, W0, W1, w2cat, gate_w, gb)
